# Initial kernel scaffold; baseline (speedup 1.0000x reference)
import jax
import jax.numpy as jnp
from jax.experimental import pallas as pl


def kernel(vox_feat, vox_coords, pos, px, W0, b0, g0, be0, W1, b1, g1, be1, Wf, bf, gf, bef, Pa, ba, ga, bea, Pb, bb, gb, beb, Pc, bc, gc, bec, KW, gact, bact, KP):
    raise NotImplementedError("write your pallas kernel here")


# speedup vs baseline: 3.3183x; 3.3183x over previous
import numpy as np
import jax
import jax.numpy as jnp
from jax.experimental import pallas as pl
from jax.experimental.pallas import tpu as pltpu

N_PTS = 16384
N_VOX = 2048
PIN = 32
VIN = 16
STEM = 64
PCH = 64
VS = 4.0
KN = 32
K = 60
RADIUS = float(np.sqrt(3.0 * VS ** 2))
EXT = RADIUS / K
R2 = np.float32(RADIUS ** 2)
EXTF = np.float32(EXT)
INF = np.float32(1.0e30)
VALID_TH = np.float32(9.0e29)
BIGI = np.int32(1 << 22)

HIGHEST = jax.lax.Precision.HIGHEST
HIGH = jax.lax.Precision.HIGH

D_ROWS = 128
C_ROWS = 256


def _dot(a, b, precision):
    return jax.lax.dot_general(a, b, (((1,), (0,)), ((), ())),
                               precision=precision,
                               preferred_element_type=jnp.float32)


def _bn(h, g, b):
    m = jnp.mean(h, axis=0, keepdims=True)
    v = jnp.mean((h - m) ** 2, axis=0, keepdims=True)
    return (h - m) / jnp.sqrt(v + 1e-5) * g + b


def _mlp3_kernel(x_ref, wa_ref, ba_ref, ga_ref, bea_ref, wb_ref, bb_ref,
                 gb_ref, beb_ref, wc_ref, bc_ref, gc_ref, bec_ref, out_ref):
    h = _bn(jnp.maximum(_dot(x_ref[...], wa_ref[...], HIGHEST) + ba_ref[...], 0.0),
            ga_ref[...], bea_ref[...])
    h = _bn(jnp.maximum(_dot(h, wb_ref[...], HIGHEST) + bb_ref[...], 0.0),
            gb_ref[...], beb_ref[...])
    out_ref[...] = _bn(jnp.maximum(_dot(h, wc_ref[...], HIGHEST) + bc_ref[...], 0.0),
                       gc_ref[...], bec_ref[...])


def _mlp3(x, wa, ba, ga, bea, wb, bb, gb, beb, wc, bc, gc, bec, out_dim):
    args = [x, wa, ba.reshape(1, -1), ga.reshape(1, -1), bea.reshape(1, -1),
            wb, bb.reshape(1, -1), gb.reshape(1, -1), beb.reshape(1, -1),
            wc, bc.reshape(1, -1), gc.reshape(1, -1), bec.reshape(1, -1)]
    return pl.pallas_call(
        _mlp3_kernel,
        out_shape=jax.ShapeDtypeStruct((x.shape[0], out_dim), jnp.float32),
    )(*args)


def _knn_kernel(vcoords_ref, posT_ref, centers_ref):
    vc = vcoords_ref[:, 1:4].astype(jnp.float32)
    p0 = posT_ref[0:1, :]
    p1 = posT_ref[1:2, :]
    p2 = posT_ref[2:3, :]
    vq0 = jnp.floor(p0 / VS)
    vq1 = jnp.floor(p1 / VS)
    vq2 = jnp.floor(p2 / VS)
    vqs = vq0 * vq0 + vq1 * vq1 + vq2 * vq2
    c0 = vc[:, 0:1]
    c1 = vc[:, 1:2]
    c2 = vc[:, 2:3]
    vcs = c0 * c0 + c1 * c1 + c2 * c2
    vqT = jnp.concatenate([vq0, vq1, vq2], axis=0)
    mm = _dot(vc.astype(jnp.bfloat16), vqT.astype(jnp.bfloat16), None)
    d2 = (vcs + vqs) - 2.0 * mm
    rmin = jnp.min(d2, axis=1, keepdims=True)
    ii = jax.lax.broadcasted_iota(jnp.int32, d2.shape, 1)
    sel = jnp.min(jnp.where(d2 == rmin, ii, BIGI), axis=1, keepdims=True)
    oh = ii == sel
    s0 = jnp.sum(jnp.where(oh, p0, 0.0), axis=1, keepdims=True)
    s1 = jnp.sum(jnp.where(oh, p1, 0.0), axis=1, keepdims=True)
    s2 = jnp.sum(jnp.where(oh, p2, 0.0), axis=1, keepdims=True)
    centers_ref[...] = jnp.concatenate([s0, s1, s2], axis=1)


def _kpconv_kernel(centers_ref, posT_ref, pf_ref, kpT_ref, weighted_ref, m_scr):
    c = centers_ref[...]
    c0 = c[:, 0:1]
    c1 = c[:, 1:2]
    c2 = c[:, 2:3]
    p0 = posT_ref[0:1, :]
    p1 = posT_ref[1:2, :]
    p2 = posT_ref[2:3, :]
    ps = p0 * p0 + p1 * p1 + p2 * p2
    cs = c0 * c0 + c1 * c1 + c2 * c2
    mm = c0 * p0 + c1 * p1 + c2 * p2
    d2c = (cs + ps) - 2.0 * mm
    mask = d2c <= R2
    m_scr[...] = jnp.where(mask, d2c, INF)
    weighted_ref[...] = jnp.zeros_like(weighted_ref)
    cnt = jnp.sum(mask.astype(jnp.float32), axis=1)
    nmax = jnp.minimum(jnp.max(cnt), float(KN)).astype(jnp.int32)

    kp0 = kpT_ref[0:1, 0:K]
    kp1 = kpT_ref[1:2, 0:K]
    kp2 = kpT_ref[2:3, 0:K]
    pf = pf_ref[...]

    def body(k, carry):
        masked = m_scr[...]
        rmin = jnp.min(masked, axis=1, keepdims=True)
        valid = rmin < VALID_TH
        ii = jax.lax.broadcasted_iota(jnp.int32, masked.shape, 1)
        sel = jnp.min(jnp.where((masked == rmin) & valid, ii, BIGI),
                      axis=1, keepdims=True)
        oh = ii == sel
        s0 = jnp.sum(jnp.where(oh, p0, 0.0), axis=1, keepdims=True)
        s1 = jnp.sum(jnp.where(oh, p1, 0.0), axis=1, keepdims=True)
        s2 = jnp.sum(jnp.where(oh, p2, 0.0), axis=1, keepdims=True)
        m_scr[...] = jnp.where(oh, INF, masked)
        ohf = jnp.where(oh, 1.0, 0.0)
        nx = _dot(ohf, pf, HIGHEST)
        d0 = s0 - c0
        d1 = s1 - c1
        d2_ = s2 - c2
        t0 = d0 - kp0
        t1 = d1 - kp1
        t2 = d2_ - kp2
        sq = t0 * t0 + t1 * t1 + t2 * t2
        wgt = jnp.maximum(1.0 - jnp.sqrt(sq + 1e-12) / EXTF, 0.0)
        weighted_ref[...] += wgt[:, :, None] * nx[:, None, :]
        return carry

    jax.lax.fori_loop(0, nmax, body, 0)


def _pconv_kernel(w2_ref, kw2_ref, out_ref):
    out_ref[...] = jnp.maximum(_dot(w2_ref[...], kw2_ref[...], HIGHEST), 0.0)


def _bn_kernel(h_ref, gact_ref, bact_ref, out_ref):
    out_ref[...] = _bn(h_ref[...], gact_ref[...], bact_ref[...])


def kernel(vox_feat, vox_coords, pos, px, W0, b0, g0, be0, W1, b1, g1, be1,
           Wf, bf, gf, bef, Pa, ba, ga, bea, Pb, bb, gb, beb, Pc, bc, gc, bec,
           KW, gact, bact, KP):
    posT = jnp.pad(pos.T, ((0, 5), (0, 0)))
    kpT = jnp.pad(KP.T, ((0, 5), (0, 68 - K)))

    xf = _mlp3(vox_feat, W0, b0, g0, be0, W1, b1, g1, be1, Wf, bf, gf, bef, STEM)
    pf = _mlp3(px, Pa, ba, ga, bea, Pb, bb, gb, beb, Pc, bc, gc, bec, PCH)

    centers = pl.pallas_call(
        _knn_kernel,
        grid=(N_VOX // C_ROWS,),
        in_specs=[
            pl.BlockSpec((C_ROWS, 4), lambda i: (i, 0)),
            pl.BlockSpec((8, N_PTS), lambda i: (0, 0)),
        ],
        out_specs=pl.BlockSpec((C_ROWS, 3), lambda i: (i, 0)),
        out_shape=jax.ShapeDtypeStruct((N_VOX, 3), jnp.float32),
    )(vox_coords, posT)

    weighted = pl.pallas_call(
        _kpconv_kernel,
        grid=(N_VOX // D_ROWS,),
        in_specs=[
            pl.BlockSpec((D_ROWS, 3), lambda i: (i, 0)),
            pl.BlockSpec((8, N_PTS), lambda i: (0, 0)),
            pl.BlockSpec((N_PTS, PCH), lambda i: (0, 0)),
            pl.BlockSpec((8, 68), lambda i: (0, 0)),
        ],
        out_specs=pl.BlockSpec((D_ROWS, K, PCH), lambda i: (i, 0, 0)),
        out_shape=jax.ShapeDtypeStruct((N_VOX, K, PCH), jnp.float32),
        scratch_shapes=[pltpu.VMEM((D_ROWS, N_PTS), jnp.float32)],
    )(centers, posT, pf, kpT)

    E_ROWS = 256
    pconv_relu = pl.pallas_call(
        _pconv_kernel,
        grid=(N_VOX // E_ROWS,),
        in_specs=[
            pl.BlockSpec((E_ROWS, K * PCH), lambda i: (i, 0)),
            pl.BlockSpec((K * PCH, PCH), lambda i: (0, 0)),
        ],
        out_specs=pl.BlockSpec((E_ROWS, PCH), lambda i: (i, 0)),
        out_shape=jax.ShapeDtypeStruct((N_VOX, PCH), jnp.float32),
    )(weighted.reshape(N_VOX, K * PCH), KW.reshape(K * PCH, PCH))

    pconv = pl.pallas_call(
        _bn_kernel,
        out_shape=jax.ShapeDtypeStruct((N_VOX, PCH), jnp.float32),
    )(pconv_relu, gact.reshape(1, -1), bact.reshape(1, -1))

    return (xf, pf, pconv)
